# baseline (speedup 1.0000x reference)
import functools

import jax
import jax.numpy as jnp
from jax import lax
from jax.experimental import pallas as pl
from jax.experimental.pallas import tpu as pltpu
from jax.experimental.pallas import tpu_sc as plsc

VOCAB = 1000000
BOTTLENECK = 32
HIDDEN = 128
B, L = 4096, 200
N = B * L

NC, NS = 2, 16
NW = NC * NS
ROWS_PER_W = N // NW
IDX_W = 128
K_INFLIGHT = 8
CHUNK = K_INFLIGHT * IDX_W
N_OUTER = ROWS_PER_W // CHUNK
IDX_ROWS_PER_W = ROWS_PER_W // IDX_W


def _sc_gather(table, idx2d):
    mesh = plsc.VectorSubcoreMesh(core_axis_name="c", subcore_axis_name="s")

    @functools.partial(
        pl.kernel,
        mesh=mesh,
        compiler_params=pltpu.CompilerParams(use_tc_tiling_on_sc=False),
        out_type=jax.ShapeDtypeStruct((N, BOTTLENECK), jnp.float32),
        scratch_types=[
            pltpu.VMEM((K_INFLIGHT, IDX_W), jnp.int32),
            pltpu.VMEM((CHUNK, BOTTLENECK), jnp.float32),
            pltpu.SemaphoreType.DMA,
        ],
    )
    def k(table_hbm, idx_hbm, out_hbm, idx_v, rows_v, sem):
        wid = lax.axis_index("s") * NC + lax.axis_index("c")

        def body(it, carry):
            idx_row_base = wid * IDX_ROWS_PER_W + it * K_INFLIGHT
            row_base = wid * ROWS_PER_W + it * CHUNK
            pltpu.sync_copy(idx_hbm.at[pl.ds(idx_row_base, K_INFLIGHT)], idx_v)
            handles = []
            for j in range(K_INFLIGHT):
                handles.append(pltpu.async_copy(
                    table_hbm.at[idx_v.at[j]],
                    rows_v.at[pl.ds(j * IDX_W, IDX_W)],
                    sem,
                ))
            for h in handles:
                h.wait()
            pltpu.sync_copy(rows_v, out_hbm.at[pl.ds(row_base, CHUNK)])
            return carry

        lax.fori_loop(0, N_OUTER, body, 0)

    return k(table, idx2d)


def _mm_body(x_ref, w_ref, o_ref):
    o_ref[...] = lax.dot_general(
        x_ref[...].astype(jnp.bfloat16), w_ref[...],
        dimension_numbers=(((1,), (0,)), ((), ())),
        preferred_element_type=jnp.float32,
    )


def _tc_expand_packed(xp, W_bd):
    BM = 512
    M = N // 4
    return pl.pallas_call(
        _mm_body,
        grid=(M // BM,),
        in_specs=[
            pl.BlockSpec((BM, 4 * BOTTLENECK), lambda i: (i, 0)),
            pl.BlockSpec((4 * BOTTLENECK, 4 * HIDDEN), lambda i: (0, 0)),
        ],
        out_specs=pl.BlockSpec((BM, 4 * HIDDEN), lambda i: (i, 0)),
        out_shape=jax.ShapeDtypeStruct((M, 4 * HIDDEN), jnp.float32),
    )(xp, W_bd)


def kernel(input_ids, table, W):
    idx2d = input_ids.reshape(N // IDX_W, IDX_W).astype(jnp.int32)
    compressed = _sc_gather(table, idx2d)
    xp = compressed.reshape(N // 4, 4 * BOTTLENECK)
    Wt = W.T.astype(jnp.bfloat16)
    W_bd = jax.scipy.linalg.block_diag(Wt, Wt, Wt, Wt)
    expanded = _tc_expand_packed(xp, W_bd)
    return expanded.reshape(B, L, HIDDEN)


# speedup vs baseline: 2.1280x; 2.1280x over previous
import functools

import jax
import jax.numpy as jnp
from jax import lax
from jax.experimental import pallas as pl
from jax.experimental.pallas import tpu as pltpu
from jax.experimental.pallas import tpu_sc as plsc

VOCAB = 1000000
BOTTLENECK = 32
HIDDEN = 128
B, L = 4096, 200
N = B * L

NC, NS = 2, 16
NW = NC * NS
ROWS_PER_W = N // NW
IDX_W = 128
IDX_ROWS_PER_W = ROWS_PER_W // IDX_W

IDX_BLOCK = 40
K_INFLIGHT = 5
N_INNER = IDX_BLOCK // K_INFLIGHT
CHUNK = K_INFLIGHT * IDX_W
N_OUTER = IDX_ROWS_PER_W // IDX_BLOCK


def _tc_expand(tT, Wt):
    BN = 4096
    grid = (VOCAB + BN - 1) // BN

    def body(t_ref, w_ref, o_ref):
        o_ref[...] = lax.dot_general(
            t_ref[...].astype(jnp.bfloat16), w_ref[...],
            dimension_numbers=(((0,), (0,)), ((), ())),
            preferred_element_type=jnp.float32,
        )

    return pl.pallas_call(
        body,
        grid=(grid,),
        in_specs=[
            pl.BlockSpec((BOTTLENECK, BN), lambda i: (0, i)),
            pl.BlockSpec((BOTTLENECK, HIDDEN), lambda i: (0, 0)),
        ],
        out_specs=pl.BlockSpec((BN, HIDDEN), lambda i: (i, 0)),
        out_shape=jax.ShapeDtypeStruct((VOCAB, HIDDEN), jnp.float32),
    )(tT, Wt)


def _sc_gather(expanded, idx2d):
    mesh = plsc.VectorSubcoreMesh(core_axis_name="c", subcore_axis_name="s")

    @functools.partial(
        pl.kernel,
        mesh=mesh,
        compiler_params=pltpu.CompilerParams(use_tc_tiling_on_sc=False),
        out_type=jax.ShapeDtypeStruct((N, HIDDEN), jnp.float32),
        scratch_types=[
            pltpu.VMEM((IDX_BLOCK, IDX_W), jnp.int32),
            pltpu.VMEM((CHUNK, HIDDEN), jnp.float32),
            pltpu.SemaphoreType.DMA,
        ],
    )
    def k(table_hbm, idx_hbm, out_hbm, idx_v, rows_v, sem):
        wid = lax.axis_index("s") * NC + lax.axis_index("c")

        def outer(o, carry):
            idx_row_base = wid * IDX_ROWS_PER_W + o * IDX_BLOCK
            pltpu.sync_copy(idx_hbm.at[pl.ds(idx_row_base, IDX_BLOCK)], idx_v)

            def inner(io, c2):
                handles = []
                for j in range(K_INFLIGHT):
                    handles.append(pltpu.async_copy(
                        table_hbm.at[idx_v.at[io * K_INFLIGHT + j]],
                        rows_v.at[pl.ds(j * IDX_W, IDX_W)],
                        sem,
                    ))
                for h in handles:
                    h.wait()
                row_base = (wid * ROWS_PER_W + o * IDX_BLOCK * IDX_W
                            + io * CHUNK)
                pltpu.sync_copy(rows_v, out_hbm.at[pl.ds(row_base, CHUNK)])
                return c2

            lax.fori_loop(0, N_INNER, inner, 0)
            return carry

        lax.fori_loop(0, N_OUTER, outer, 0)

    return k(expanded, idx2d)


def kernel(input_ids, table, W):
    idx2d = input_ids.reshape(N // IDX_W, IDX_W).astype(jnp.int32)
    tT = table.T
    Wt = W.T.astype(jnp.bfloat16)
    expanded = _tc_expand(tT, Wt)
    out = _sc_gather(expanded, idx2d)
    return out.reshape(B, L, HIDDEN)


# speedup vs baseline: 2.1801x; 1.0245x over previous
import functools

import jax
import jax.numpy as jnp
from jax import lax
from jax.experimental import pallas as pl
from jax.experimental.pallas import tpu as pltpu
from jax.experimental.pallas import tpu_sc as plsc

VOCAB = 1000000
BOTTLENECK = 32
HIDDEN = 128
B, L = 4096, 200
N = B * L

NC, NS = 2, 16
NW = NC * NS
ROWS_PER_W = N // NW
IDX_W = 128
IDX_ROWS_PER_W = ROWS_PER_W // IDX_W

IDX_BLOCK = 40
G_GATHERS = 2
BUFROWS = G_GATHERS * IDX_W
BUFBYTES = BUFROWS * HIDDEN * 4
GROUPS_PER_BLOCK = IDX_BLOCK // G_GATHERS
N_OUTER = IDX_ROWS_PER_W // IDX_BLOCK


def _tc_expand(tT, Wt):
    BN = 4096
    grid = (VOCAB + BN - 1) // BN

    def body(t_ref, w_ref, o_ref):
        o_ref[...] = lax.dot_general(
            t_ref[...].astype(jnp.bfloat16), w_ref[...],
            dimension_numbers=(((0,), (0,)), ((), ())),
            preferred_element_type=jnp.float32,
        )

    return pl.pallas_call(
        body,
        grid=(grid,),
        in_specs=[
            pl.BlockSpec((BOTTLENECK, BN), lambda i: (0, i)),
            pl.BlockSpec((BOTTLENECK, HIDDEN), lambda i: (0, 0)),
        ],
        out_specs=pl.BlockSpec((BN, HIDDEN), lambda i: (i, 0)),
        out_shape=jax.ShapeDtypeStruct((VOCAB, HIDDEN), jnp.float32),
    )(tT, Wt)


def _sc_gather(expanded, idx2d):
    mesh = plsc.VectorSubcoreMesh(core_axis_name="c", subcore_axis_name="s")

    @functools.partial(
        pl.kernel,
        mesh=mesh,
        compiler_params=pltpu.CompilerParams(use_tc_tiling_on_sc=False),
        out_type=jax.ShapeDtypeStruct((N, HIDDEN), jnp.float32),
        scratch_types=[
            pltpu.VMEM((IDX_BLOCK, IDX_W), jnp.int32),
            pltpu.VMEM((2 * BUFROWS, HIDDEN), jnp.float32),
            pltpu.SemaphoreType.DMA,
            pltpu.SemaphoreType.DMA,
        ],
    )
    def k(table_hbm, idx_hbm, out_hbm, idx_v, rows_v, gsem, wsem):
        wid = lax.axis_index("s") * NC + lax.axis_index("c")

        def outer(o, carry):
            idx_row_base = wid * IDX_ROWS_PER_W + o * IDX_BLOCK
            pltpu.sync_copy(idx_hbm.at[pl.ds(idx_row_base, IDX_BLOCK)], idx_v)

            def inner(g, c2):
                i_glob = g + o * GROUPS_PER_BLOCK
                p = i_glob % 2
                buf = rows_v.at[pl.ds(p * BUFROWS, BUFROWS)]

                def _drain(_):
                    pltpu.make_async_copy(
                        out_hbm.at[pl.ds(0, BUFROWS)], buf, wsem).wait()
                    return 0

                lax.cond(i_glob >= 2, _drain, lambda _: 0, 0)
                handles = []
                for j in range(G_GATHERS):
                    handles.append(pltpu.async_copy(
                        table_hbm.at[idx_v.at[g * G_GATHERS + j]],
                        buf.at[pl.ds(j * IDX_W, IDX_W)],
                        gsem,
                    ))
                for h in handles:
                    h.wait()
                row_base = (wid * ROWS_PER_W + o * IDX_BLOCK * IDX_W
                            + g * BUFROWS)
                pltpu.async_copy(
                    buf, out_hbm.at[pl.ds(row_base, BUFROWS)], wsem)
                return c2

            lax.fori_loop(0, GROUPS_PER_BLOCK, inner, 0)
            return carry

        lax.fori_loop(0, N_OUTER, outer, 0)
        pltpu.make_async_copy(
            out_hbm.at[pl.ds(0, BUFROWS)],
            rows_v.at[pl.ds(0, BUFROWS)], wsem).wait()
        pltpu.make_async_copy(
            out_hbm.at[pl.ds(0, BUFROWS)],
            rows_v.at[pl.ds(BUFROWS, BUFROWS)], wsem).wait()

    return k(expanded, idx2d)


def kernel(input_ids, table, W):
    idx2d = input_ids.reshape(N // IDX_W, IDX_W).astype(jnp.int32)
    tT = table.T
    Wt = W.T.astype(jnp.bfloat16)
    expanded = _tc_expand(tT, Wt)
    out = _sc_gather(expanded, idx2d)
    return out.reshape(B, L, HIDDEN)


# speedup vs baseline: 2.1890x; 1.0041x over previous
import functools

import jax
import jax.numpy as jnp
from jax import lax
from jax.experimental import pallas as pl
from jax.experimental.pallas import tpu as pltpu
from jax.experimental.pallas import tpu_sc as plsc

VOCAB = 1000000
BOTTLENECK = 32
HIDDEN = 128
B, L = 4096, 200
N = B * L

NC, NS = 2, 16
NW = NC * NS
ROWS_PER_W = N // NW
IDX_W = 128
IDX_ROWS_PER_W = ROWS_PER_W // IDX_W

NSLOT = 6
LAG = 3
BUFROWS = IDX_W
N_STEPS = IDX_ROWS_PER_W


def _tc_expand(tT, Wt):
    BN = 4096
    grid = (VOCAB + BN - 1) // BN

    def body(t_ref, w_ref, o_ref):
        o_ref[...] = lax.dot_general(
            t_ref[...].astype(jnp.bfloat16), w_ref[...],
            dimension_numbers=(((0,), (0,)), ((), ())),
            preferred_element_type=jnp.float32,
        )

    return pl.pallas_call(
        body,
        grid=(grid,),
        in_specs=[
            pl.BlockSpec((BOTTLENECK, BN), lambda i: (0, i)),
            pl.BlockSpec((BOTTLENECK, HIDDEN), lambda i: (0, 0)),
        ],
        out_specs=pl.BlockSpec((BN, HIDDEN), lambda i: (i, 0)),
        out_shape=jax.ShapeDtypeStruct((VOCAB, HIDDEN), jnp.float32),
    )(tT, Wt)


def _sc_gather(expanded, idx2d):
    mesh = plsc.VectorSubcoreMesh(core_axis_name="c", subcore_axis_name="s")

    @functools.partial(
        pl.kernel,
        mesh=mesh,
        compiler_params=pltpu.CompilerParams(use_tc_tiling_on_sc=False),
        out_type=jax.ShapeDtypeStruct((N, HIDDEN), jnp.float32),
        scratch_types=[
            pltpu.VMEM((IDX_ROWS_PER_W, IDX_W), jnp.int32),
            pltpu.VMEM((NSLOT * BUFROWS, HIDDEN), jnp.float32),
            pltpu.SemaphoreType.DMA,
            pltpu.SemaphoreType.DMA,
        ],
    )
    def k(table_hbm, idx_hbm, out_hbm, idx_v, rows_v, gsem, wsem):
        wid = lax.axis_index("s") * NC + lax.axis_index("c")
        row0 = wid * ROWS_PER_W
        pltpu.sync_copy(
            idx_hbm.at[pl.ds(wid * IDX_ROWS_PER_W, IDX_ROWS_PER_W)], idx_v)

        def slot(i):
            return rows_v.at[pl.ds((i % NSLOT) * BUFROWS, BUFROWS)]

        def fire(i):
            pltpu.async_copy(table_hbm.at[idx_v.at[i]], slot(i), gsem)

        def wait_gather(i):
            pltpu.make_async_copy(table_hbm.at[idx_v.at[i]], slot(i),
                                  gsem).wait()

        def write(i):
            pltpu.async_copy(
                slot(i), out_hbm.at[pl.ds(row0 + i * BUFROWS, BUFROWS)],
                wsem)

        def drain_write(i):
            pltpu.make_async_copy(
                out_hbm.at[pl.ds(0, BUFROWS)], slot(i), wsem).wait()

        for j in range(LAG):
            fire(j)

        def body(i, carry):
            wait_gather(i)
            write(i)

            def _reuse(_):
                lax.cond(i >= LAG, lambda __: (drain_write(i - LAG), 0)[1],
                         lambda __: 0, 0)
                fire(i + LAG)
                return 0

            lax.cond(i + LAG < N_STEPS, _reuse, lambda _: 0, 0)
            return carry

        lax.fori_loop(0, N_STEPS, body, 0)
        for j in range(NSLOT):
            drain_write(N_STEPS - NSLOT + j)

    return k(expanded, idx2d)


def kernel(input_ids, table, W):
    idx2d = input_ids.reshape(N // IDX_W, IDX_W).astype(jnp.int32)
    tT = table.T
    Wt = W.T.astype(jnp.bfloat16)
    expanded = _tc_expand(tT, Wt)
    out = _sc_gather(expanded, idx2d)
    return out.reshape(B, L, HIDDEN)


# speedup vs baseline: 2.4864x; 1.1359x over previous
import functools

import jax
import jax.numpy as jnp
from jax import lax
from jax.experimental import pallas as pl
from jax.experimental.pallas import tpu as pltpu
from jax.experimental.pallas import tpu_sc as plsc

VOCAB = 1000000
BOTTLENECK = 32
HIDDEN = 128
B, L = 4096, 200
N = B * L

NC, NS = 2, 16
NW = NC * NS
ROWS_PER_W = N // NW
IDX_W = 128
IDX_ROWS_PER_W = ROWS_PER_W // IDX_W

NSLOT = 6
LAG = 4
BUFROWS = IDX_W
N_STEPS = IDX_ROWS_PER_W


def _tc_expand(tT, Wt):
    BN = 8192
    grid = (VOCAB + BN - 1) // BN

    def body(t_ref, w_ref, o_ref):
        o_ref[...] = lax.dot_general(
            t_ref[...].astype(jnp.bfloat16), w_ref[...],
            dimension_numbers=(((0,), (0,)), ((), ())),
            preferred_element_type=jnp.float32,
        )

    return pl.pallas_call(
        body,
        grid=(grid,),
        in_specs=[
            pl.BlockSpec((BOTTLENECK, BN), lambda i: (0, i)),
            pl.BlockSpec((BOTTLENECK, HIDDEN), lambda i: (0, 0)),
        ],
        out_specs=pl.BlockSpec((BN, HIDDEN), lambda i: (i, 0)),
        out_shape=jax.ShapeDtypeStruct((VOCAB, HIDDEN), jnp.float32),
    )(tT, Wt)


def _sc_gather(expanded, idx2d):
    mesh = plsc.VectorSubcoreMesh(core_axis_name="c", subcore_axis_name="s")

    @functools.partial(
        pl.kernel,
        mesh=mesh,
        compiler_params=pltpu.CompilerParams(use_tc_tiling_on_sc=False),
        out_type=jax.ShapeDtypeStruct((N, HIDDEN), jnp.float32),
        scratch_types=[
            pltpu.VMEM((IDX_ROWS_PER_W, IDX_W), jnp.int32),
            pltpu.VMEM((NSLOT * BUFROWS, HIDDEN), jnp.float32),
            pltpu.SemaphoreType.DMA,
            pltpu.SemaphoreType.DMA,
        ],
    )
    def k(table_hbm, idx_hbm, out_hbm, idx_v, rows_v, gsem, wsem):
        wid = lax.axis_index("s") * NC + lax.axis_index("c")
        row0 = wid * ROWS_PER_W
        pltpu.sync_copy(
            idx_hbm.at[pl.ds(wid * IDX_ROWS_PER_W, IDX_ROWS_PER_W)], idx_v)

        def slot(i):
            return rows_v.at[pl.ds((i % NSLOT) * BUFROWS, BUFROWS)]

        def fire(i):
            pltpu.async_copy(table_hbm.at[idx_v.at[i]], slot(i), gsem)

        def wait_gather(i):
            pltpu.make_async_copy(table_hbm.at[idx_v.at[i]], slot(i),
                                  gsem).wait()

        def write(i):
            pltpu.async_copy(
                slot(i), out_hbm.at[pl.ds(row0 + i * BUFROWS, BUFROWS)],
                wsem)

        def drain_write(i):
            pltpu.make_async_copy(
                out_hbm.at[pl.ds(0, BUFROWS)], slot(i), wsem).wait()

        for j in range(LAG):
            fire(j)

        def body(i, carry):
            wait_gather(i)
            write(i)

            def _reuse(_):
                lax.cond(i >= NSLOT - LAG,
                         lambda __: (drain_write(i), 0)[1],
                         lambda __: 0, 0)
                fire(i + LAG)
                return 0

            lax.cond(i + LAG < N_STEPS, _reuse, lambda _: 0, 0)
            return carry

        lax.fori_loop(0, N_STEPS, body, 0)
        for j in range(NSLOT):
            drain_write(N_STEPS - NSLOT + j)

    return k(expanded, idx2d)


def kernel(input_ids, table, W):
    idx2d = input_ids.reshape(N // IDX_W, IDX_W).astype(jnp.int32)
    tT = table.T
    Wt = W.T.astype(jnp.bfloat16)
    expanded = _tc_expand(tT, Wt)
    out = _sc_gather(expanded, idx2d)
    return out.reshape(B, L, HIDDEN)
